# baseline (speedup 1.0000x reference)
import functools

import jax
import jax.numpy as jnp
from jax import lax
from jax.experimental import pallas as pl
from jax.experimental.pallas import tpu as pltpu
from jax.experimental.pallas import tpu_sc as plsc

_NC = 2
_NS = 16
_NW = _NC * _NS
_LANES = 16


def _make_kernel(B, L, D, V):
    assert B % (_NW * 128) == 0 or B == _NW * 128
    bpw = B // _NW
    assert bpw == 128
    ch = 8
    nch = bpw // ch
    assert L % 8 == 0
    ltr = L // 8
    lpad = 16 * ((L + 15) // 16)
    assert D == 2 * _LANES

    mesh = plsc.VectorSubcoreMesh(core_axis_name="c", subcore_axis_name="s")

    @functools.partial(
        pl.kernel,
        mesh=mesh,
        out_type=jax.ShapeDtypeStruct((B, D), jnp.float32),
        scratch_types=[
            pltpu.VMEM((ltr, 8, 128), jnp.int32),
            pltpu.VMEM((bpw, lpad), jnp.int32),
            pltpu.VMEM((ch, L, D), jnp.float32),
            pltpu.VMEM((ch, D), jnp.float32),
            pltpu.SemaphoreType.DMA,
        ],
        compiler_params=pltpu.CompilerParams(
            use_tc_tiling_on_sc=False, needs_layout_passes=False),
    )
    def k(idx4_hbm, table_hbm, out_hbm, idx_v, packed_v, rows_v, out_v, sem):
        wid = lax.axis_index("s") * _NC + lax.axis_index("c")
        scale = jnp.float32(1.0 / L)
        lane = lax.iota(jnp.int32, 16)

        pltpu.sync_copy(idx4_hbm.at[:, wid], idx_v)

        def repack_b(b, _):
            bcol = jnp.broadcast_to(b, (16,)).astype(jnp.int32)
            for j in range(lpad // 16):
                l = jnp.minimum(lane + (16 * j), L - 1)
                v = plsc.load_gather(
                    idx_v, [l >> 3, jnp.bitwise_and(l, 7), bcol])
                packed_v[b, pl.ds(16 * j, 16)] = v
            return _

        lax.fori_loop(0, bpw, repack_b, 0)

        def chunk_body(c, _):
            row0 = wid * bpw + c * ch
            copies = []
            for b in range(ch):
                copies.append(pltpu.async_copy(
                    table_hbm.at[packed_v.at[c * ch + b, pl.ds(0, L)]],
                    rows_v.at[b],
                    sem,
                ))
            for cp in copies:
                cp.wait()
            lh = L // 2
            for b in range(ch):
                def red(r, carry):
                    a0, a1, a2, a3 = carry
                    a0 = a0 + rows_v[b, r, pl.ds(0, _LANES)]
                    a1 = a1 + rows_v[b, r, pl.ds(_LANES, _LANES)]
                    a2 = a2 + rows_v[b, r + lh, pl.ds(0, _LANES)]
                    a3 = a3 + rows_v[b, r + lh, pl.ds(_LANES, _LANES)]
                    return a0, a1, a2, a3
                z = jnp.zeros((_LANES,), jnp.float32)
                a0, a1, a2, a3 = lax.fori_loop(0, lh, red, (z, z, z, z))
                out_v[b, pl.ds(0, _LANES)] = (a0 + a2) * scale
                out_v[b, pl.ds(_LANES, _LANES)] = (a1 + a3) * scale
            pltpu.sync_copy(out_v, out_hbm.at[pl.ds(row0, ch)])
            return _

        lax.fori_loop(0, nch, chunk_body, 0)

    return k


def kernel(inputs, table):
    B, L = inputs.shape
    V, D = table.shape
    idx4 = (
        inputs.astype(jnp.int32)
        .T.reshape(L // 8, 8, B // 128, 128)
        .transpose(0, 2, 1, 3)
    )
    return _make_kernel(B, L, D, V)(idx4, table)
